# Initial kernel scaffold; baseline (speedup 1.0000x reference)
import jax
import jax.numpy as jnp
from jax.experimental import pallas as pl


def kernel(xyzs, feature, ln_g, ln_b, W_qkv, W_sp, W_out, b_out):
    raise NotImplementedError("write your pallas kernel here")


# speedup vs baseline: 21.6484x; 21.6484x over previous
import math

import jax
import jax.numpy as jnp
from jax.experimental import pallas as pl
from jax.experimental.pallas import tpu as pltpu

_B, _L, _N = 4, 4, 1024
_DIM, _HEADS, _DH = 256, 8, 32
_INNER = _HEADS * _DH
_NS = 8
_R2 = 0.2 * 0.2
_SCALE = _DH ** -0.5


def _ln_qkv_body(f_ref, g_ref, b_ref, wt_ref, kv_ref, q_ref):
    x = f_ref[0]
    mu = jnp.mean(x, axis=-1, keepdims=True)
    var = jnp.mean((x - mu) ** 2, axis=-1, keepdims=True)
    nf = (x - mu) / jnp.sqrt(var + 1e-5) * g_ref[0] + b_ref[0]
    qkv = jnp.dot(nf.astype(jnp.bfloat16), wt_ref[...],
                  preferred_element_type=jnp.float32).astype(jnp.bfloat16)
    q_ref[0] = qkv[:, 0:_INNER]
    kv_ref[0] = qkv[:, _INNER:3 * _INNER]


def _attn_body(xyz_ref, xyzt_ref, kv_ref, q3_ref, wsp_ref, wot_ref, bo_ref,
               o_ref, gm_ref, cnt_ref, xs16_ref, fbsc_ref, fbv_ref, fbx_ref,
               m_ref, z_ref, av_ref, da_ref):
    q3 = q3_ref[0, 0].astype(jnp.float32)
    xyz = xyz_ref[0]
    xq = xyz[_L - 1]
    sqq = jnp.sum(xq * xq, axis=-1, keepdims=True)

    ii = jax.lax.broadcasted_iota(jnp.int32, (_N, _N), 0)
    jj = jax.lax.broadcasted_iota(jnp.int32, (_N, _N), 1)
    tri16 = jnp.where(ii <= jj, 1.0, 0.0).astype(jnp.bfloat16)
    seg = (jax.lax.broadcasted_iota(jnp.int32, (_DIM, _HEADS), 0) // _DH ==
           jax.lax.broadcasted_iota(jnp.int32, (_DIM, _HEADS), 1)
           ).astype(jnp.float32)
    seg_t = seg.T

    for l in range(_L):
        xs = xyz[l]
        sqs = jnp.sum(xs * xs, axis=-1)[None, :]
        d2 = (sqq + sqs) - 2.0 * jnp.dot(xq, xyzt_ref[0, l],
                                         preferred_element_type=jnp.float32)
        mask16 = jnp.where(d2 < _R2, 1.0, 0.0).astype(jnp.bfloat16)
        grank = jnp.dot(mask16, tri16, preferred_element_type=jnp.float32)
        gm_ref[l] = grank * mask16.astype(jnp.float32)
        cnt_ref[l] = grank[:, _N - 1:_N]
        xs16_ref[l] = xs.astype(jnp.bfloat16)

    m_ref[...] = jnp.full((_N, _HEADS), -1e30, jnp.float32)
    z_ref[...] = jnp.zeros((_N, _HEADS), jnp.float32)
    av_ref[...] = jnp.zeros((_N, _INNER), jnp.float32)

    def body(s, carry):
        l = s // _NS
        si = s % _NS
        sif = si.astype(jnp.float32)
        gml = gm_ref[l]
        oh16 = jnp.where(gml == sif + 1.0, 1.0, 0.0).astype(jnp.bfloat16)
        kvl = kv_ref[0, l]
        xsl = xs16_ref[l]
        g = jnp.dot(oh16, kvl, preferred_element_type=jnp.float32)
        xg_raw = jnp.dot(oh16, xsl, preferred_element_type=jnp.float32)
        kg = g[:, 0:_INNER]
        v_raw = g[:, _INNER:2 * _INNER]
        sc_raw = jnp.dot(kg * q3, seg,
                         preferred_element_type=jnp.float32) * _SCALE
        cntl = cnt_ref[l]

        k0 = kvl[0:1, 0:_INNER].astype(jnp.float32)
        p0sc = jnp.dot(k0 * q3, seg,
                       preferred_element_type=jnp.float32) * _SCALE
        p0v = kvl[0:1, _INNER:2 * _INNER].astype(jnp.float32)
        p0x = xsl[0:1, :].astype(jnp.float32)
        has = cntl > 0.0
        first = si == 0
        fbsc = jnp.where(first, jnp.where(has, sc_raw, p0sc), fbsc_ref[...])
        fbv = jnp.where(first, jnp.where(has, v_raw, p0v), fbv_ref[...])
        fbx = jnp.where(first, jnp.where(has, xg_raw, p0x), fbx_ref[...])
        fbsc_ref[...] = fbsc
        fbv_ref[...] = fbv
        fbx_ref[...] = fbx

        found = cntl > sif
        sc = jnp.where(found, sc_raw, fbsc)
        v = jnp.where(found, v_raw, fbv)
        xg = jnp.where(found, xg_raw, fbx)

        m_old = m_ref[...]
        m_new = jnp.maximum(m_old, sc)
        e = jnp.exp(sc - m_new)
        r = jnp.exp(m_old - m_new)
        m_ref[...] = m_new
        z_ref[...] = z_ref[...] * r + e
        r_exp = jnp.dot(r, seg_t, preferred_element_type=jnp.float32)
        e_exp = jnp.dot(e, seg_t, preferred_element_type=jnp.float32)
        av_ref[...] = av_ref[...] * r_exp + e_exp * v
        isf = s == 0
        for d in range(3):
            cur = e * (xg[:, d:d + 1] - xq[:, d:d + 1])
            da_ref[d] = jnp.where(isf, cur,
                                  jnp.maximum(da_ref[d] * r, cur))
        return carry

    jax.lax.fori_loop(0, _L * _NS, body, 0)

    inv_z = 1.0 / z_ref[...]
    av = av_ref[...] * jnp.dot(inv_z, seg_t,
                               preferred_element_type=jnp.float32)
    dis = jnp.zeros((_N, _INNER), jnp.float32)
    for d in range(3):
        dad = da_ref[d] * inv_z
        dis = dis + jnp.dot(dad, seg_t,
                            preferred_element_type=jnp.float32) * \
            wsp_ref[d:d + 1, :]
    y = jnp.dot(av + dis, wot_ref[...],
                preferred_element_type=jnp.float32) + bo_ref[0]
    o_ref[0] = y * 0.5 * (1.0 + jax.lax.erf(y * (1.0 / math.sqrt(2.0))))


def _resid_body(g_ref, f_ref, o_ref):
    o_ref[0] = g_ref[0] + f_ref[0]


@jax.jit
def kernel(xyzs, feature, ln_g, ln_b, W_qkv, W_sp, W_out, b_out):
    b, l, n, dim = feature.shape
    ff = feature.reshape(b * l, n, dim)
    kv, q = pl.pallas_call(
        _ln_qkv_body,
        grid=(b * l,),
        in_specs=[
            pl.BlockSpec((1, n, dim), lambda i: (i, 0, 0)),
            pl.BlockSpec((1, dim), lambda i: (0, 0)),
            pl.BlockSpec((1, dim), lambda i: (0, 0)),
            pl.BlockSpec((dim, 3 * _INNER), lambda i: (0, 0)),
        ],
        out_specs=[
            pl.BlockSpec((1, n, 2 * _INNER), lambda i: (i, 0, 0)),
            pl.BlockSpec((1, n, _INNER), lambda i: (i, 0, 0)),
        ],
        out_shape=[
            jax.ShapeDtypeStruct((b * l, n, 2 * _INNER), jnp.bfloat16),
            jax.ShapeDtypeStruct((b * l, n, _INNER), jnp.bfloat16),
        ],
    )(ff, ln_g.reshape(1, dim), ln_b.reshape(1, dim),
      W_qkv.T.astype(jnp.bfloat16))

    xyzs_f = xyzs.reshape(b, l, n, 3)
    xyzs_t = jnp.swapaxes(xyzs_f, 2, 3)
    wsp_tiled = jnp.tile(W_sp.T, (1, _HEADS))
    g_out = pl.pallas_call(
        _attn_body,
        grid=(b,),
        in_specs=[
            pl.BlockSpec((1, l, n, 3), lambda i: (i, 0, 0, 0)),
            pl.BlockSpec((1, l, 3, n), lambda i: (i, 0, 0, 0)),
            pl.BlockSpec((1, l, n, 2 * _INNER), lambda i: (i, 0, 0, 0)),
            pl.BlockSpec((1, 1, n, _INNER), lambda i: (i, l - 1, 0, 0)),
            pl.BlockSpec((3, _INNER), lambda i: (0, 0)),
            pl.BlockSpec((dim, dim), lambda i: (0, 0)),
            pl.BlockSpec((1, dim), lambda i: (0, 0)),
        ],
        out_specs=pl.BlockSpec((1, n, dim), lambda i: (i, 0, 0)),
        out_shape=jax.ShapeDtypeStruct((b, n, dim), jnp.float32),
        scratch_shapes=[
            pltpu.VMEM((l, n, n), jnp.float32),
            pltpu.VMEM((l, n, 1), jnp.float32),
            pltpu.VMEM((l, n, 3), jnp.bfloat16),
            pltpu.VMEM((n, _HEADS), jnp.float32),
            pltpu.VMEM((n, _INNER), jnp.float32),
            pltpu.VMEM((n, 3), jnp.float32),
            pltpu.VMEM((n, _HEADS), jnp.float32),
            pltpu.VMEM((n, _HEADS), jnp.float32),
            pltpu.VMEM((n, _INNER), jnp.float32),
            pltpu.VMEM((3, n, _HEADS), jnp.float32),
        ],
    )(xyzs_f, xyzs_t, kv.reshape(b, l, n, 2 * _INNER),
      q.reshape(b, l, n, _INNER), wsp_tiled, W_out.T, b_out.reshape(1, dim))

    out = pl.pallas_call(
        _resid_body,
        grid=(b * l,),
        in_specs=[
            pl.BlockSpec((1, n, dim), lambda i: (i // l, 0, 0)),
            pl.BlockSpec((1, n, dim), lambda i: (i, 0, 0)),
        ],
        out_specs=pl.BlockSpec((1, n, dim), lambda i: (i, 0, 0)),
        out_shape=jax.ShapeDtypeStruct((b * l, n, dim), jnp.float32),
    )(g_out, ff)
    return out.reshape(b, l, n, dim)


# speedup vs baseline: 22.6774x; 1.0475x over previous
import math

import jax
import jax.numpy as jnp
from jax.experimental import pallas as pl
from jax.experimental.pallas import tpu as pltpu

_B, _L, _N = 4, 4, 1024
_DIM, _HEADS, _DH = 256, 8, 32
_INNER = _HEADS * _DH
_NS = 8
_R2 = 0.2 * 0.2
_SCALE = _DH ** -0.5


def _ln_qkv_body(f_ref, g_ref, b_ref, wt_ref, kv_ref, q_ref):
    x = f_ref[0]
    mu = jnp.mean(x, axis=-1, keepdims=True)
    var = jnp.mean((x - mu) ** 2, axis=-1, keepdims=True)
    nf = (x - mu) / jnp.sqrt(var + 1e-5) * g_ref[0] + b_ref[0]
    qkv = jnp.dot(nf.astype(jnp.bfloat16), wt_ref[...],
                  preferred_element_type=jnp.float32).astype(jnp.bfloat16)
    q_ref[0] = qkv[:, 0:_INNER]
    kv_ref[0] = qkv[:, _INNER:3 * _INNER]


def _attn_body(xyz_ref, xyzt_ref, kv_ref, q3_ref, wsp_ref, wot_ref, bo_ref,
               o_ref, gm_ref, cnt_ref, kvx_ref, p0sc_ref, fbsc_ref, fbv_ref,
               fbx_ref, m_ref, z_ref, av_ref, da_ref):
    q3 = q3_ref[0, 0].astype(jnp.float32)
    xyz = xyz_ref[0]
    xq = xyz[_L - 1]
    sqq = jnp.sum(xq * xq, axis=-1, keepdims=True)

    ii = jax.lax.broadcasted_iota(jnp.int32, (_N, _N), 0)
    jj = jax.lax.broadcasted_iota(jnp.int32, (_N, _N), 1)
    tri16 = jnp.where(ii <= jj, 1.0, 0.0).astype(jnp.bfloat16)
    seg = (jax.lax.broadcasted_iota(jnp.int32, (_DIM, _HEADS), 0) // _DH ==
           jax.lax.broadcasted_iota(jnp.int32, (_DIM, _HEADS), 1)
           ).astype(jnp.float32)
    seg_t = seg.T

    for l in range(_L):
        xs = xyz[l]
        sqs = jnp.sum(xs * xs, axis=-1)[None, :]
        d2 = (sqq + sqs) - 2.0 * jnp.dot(xq, xyzt_ref[0, l],
                                         preferred_element_type=jnp.float32)
        mask16 = jnp.where(d2 < _R2, 1.0, 0.0).astype(jnp.bfloat16)
        grank = jnp.dot(mask16, tri16, preferred_element_type=jnp.float32)
        gm_ref[l] = (grank * mask16.astype(jnp.float32)).astype(jnp.bfloat16)
        cnt_ref[l] = grank[:, _N - 1:_N]
        kvx_ref[l] = jnp.concatenate(
            [kv_ref[0, l], xs.astype(jnp.bfloat16)], axis=-1)
        k0 = kv_ref[0, l, 0:1, 0:_INNER].astype(jnp.float32)
        p0sc_ref[l] = jnp.dot(k0 * q3, seg,
                              preferred_element_type=jnp.float32) * _SCALE

    m_ref[...] = jnp.full((_N, _HEADS), -1e30, jnp.float32)
    z_ref[...] = jnp.zeros((_N, _HEADS), jnp.float32)
    av_ref[...] = jnp.zeros((_N, _INNER), jnp.float32)

    def body(s, carry):
        l = s // _NS
        si = s % _NS
        sif = si.astype(jnp.float32)
        gml = gm_ref[l]
        tgt = (si + 1).astype(jnp.bfloat16)
        oh16 = jnp.where(gml == tgt, jnp.full((), 1, jnp.bfloat16),
                         jnp.full((), 0, jnp.bfloat16))
        kvxl = kvx_ref[l]
        g = jnp.dot(oh16, kvxl, preferred_element_type=jnp.float32)
        kg = g[:, 0:_INNER]
        v_raw = g[:, _INNER:2 * _INNER]
        xg_raw = g[:, 2 * _INNER:2 * _INNER + 3]
        sc_raw = jnp.dot(kg * q3, seg,
                         preferred_element_type=jnp.float32) * _SCALE
        cntl = cnt_ref[l]

        p0sc = p0sc_ref[l]
        p0v = kvxl[0:1, _INNER:2 * _INNER].astype(jnp.float32)
        p0x = kvxl[0:1, 2 * _INNER:2 * _INNER + 3].astype(jnp.float32)
        has = cntl > 0.0
        first = si == 0
        fbsc = jnp.where(first, jnp.where(has, sc_raw, p0sc), fbsc_ref[...])
        fbv = jnp.where(first, jnp.where(has, v_raw, p0v), fbv_ref[...])
        fbx = jnp.where(first, jnp.where(has, xg_raw, p0x), fbx_ref[...])
        fbsc_ref[...] = fbsc
        fbv_ref[...] = fbv
        fbx_ref[...] = fbx

        found = cntl > sif
        sc = jnp.where(found, sc_raw, fbsc)
        v = jnp.where(found, v_raw, fbv)
        xg = jnp.where(found, xg_raw, fbx)

        m_old = m_ref[...]
        m_new = jnp.maximum(m_old, sc)
        e = jnp.exp(sc - m_new)
        r = jnp.exp(m_old - m_new)
        m_ref[...] = m_new
        z_ref[...] = z_ref[...] * r + e
        r_exp = jnp.dot(r, seg_t, preferred_element_type=jnp.float32)
        e_exp = jnp.dot(e, seg_t, preferred_element_type=jnp.float32)
        av_ref[...] = av_ref[...] * r_exp + e_exp * v
        isf = s == 0
        for d in range(3):
            cur = e * (xg[:, d:d + 1] - xq[:, d:d + 1])
            da_ref[d] = jnp.where(isf, cur,
                                  jnp.maximum(da_ref[d] * r, cur))
        return carry

    jax.lax.fori_loop(0, _L * _NS, body, 0)

    inv_z = 1.0 / z_ref[...]
    av = av_ref[...] * jnp.dot(inv_z, seg_t,
                               preferred_element_type=jnp.float32)
    dis = jnp.zeros((_N, _INNER), jnp.float32)
    for d in range(3):
        dad = da_ref[d] * inv_z
        dis = dis + jnp.dot(dad, seg_t,
                            preferred_element_type=jnp.float32) * \
            wsp_ref[d:d + 1, :]
    y = jnp.dot(av + dis, wot_ref[...],
                preferred_element_type=jnp.float32) + bo_ref[0]
    o_ref[0] = y * 0.5 * (1.0 + jax.lax.erf(y * (1.0 / math.sqrt(2.0))))


def _resid_body(g_ref, f_ref, o_ref):
    o_ref[0] = g_ref[0] + f_ref[0]


@jax.jit
def kernel(xyzs, feature, ln_g, ln_b, W_qkv, W_sp, W_out, b_out):
    b, l, n, dim = feature.shape
    ff = feature.reshape(b * l, n, dim)
    kv, q = pl.pallas_call(
        _ln_qkv_body,
        grid=(b * l,),
        in_specs=[
            pl.BlockSpec((1, n, dim), lambda i: (i, 0, 0)),
            pl.BlockSpec((1, dim), lambda i: (0, 0)),
            pl.BlockSpec((1, dim), lambda i: (0, 0)),
            pl.BlockSpec((dim, 3 * _INNER), lambda i: (0, 0)),
        ],
        out_specs=[
            pl.BlockSpec((1, n, 2 * _INNER), lambda i: (i, 0, 0)),
            pl.BlockSpec((1, n, _INNER), lambda i: (i, 0, 0)),
        ],
        out_shape=[
            jax.ShapeDtypeStruct((b * l, n, 2 * _INNER), jnp.bfloat16),
            jax.ShapeDtypeStruct((b * l, n, _INNER), jnp.bfloat16),
        ],
    )(ff, ln_g.reshape(1, dim), ln_b.reshape(1, dim),
      W_qkv.T.astype(jnp.bfloat16))

    xyzs_f = xyzs.reshape(b, l, n, 3)
    xyzs_t = jnp.swapaxes(xyzs_f, 2, 3)
    wsp_tiled = jnp.tile(W_sp.T, (1, _HEADS))
    g_out = pl.pallas_call(
        _attn_body,
        grid=(b,),
        in_specs=[
            pl.BlockSpec((1, l, n, 3), lambda i: (i, 0, 0, 0)),
            pl.BlockSpec((1, l, 3, n), lambda i: (i, 0, 0, 0)),
            pl.BlockSpec((1, l, n, 2 * _INNER), lambda i: (i, 0, 0, 0)),
            pl.BlockSpec((1, 1, n, _INNER), lambda i: (i, l - 1, 0, 0)),
            pl.BlockSpec((3, _INNER), lambda i: (0, 0)),
            pl.BlockSpec((dim, dim), lambda i: (0, 0)),
            pl.BlockSpec((1, dim), lambda i: (0, 0)),
        ],
        out_specs=pl.BlockSpec((1, n, dim), lambda i: (i, 0, 0)),
        out_shape=jax.ShapeDtypeStruct((b, n, dim), jnp.float32),
        scratch_shapes=[
            pltpu.VMEM((l, n, n), jnp.bfloat16),
            pltpu.VMEM((l, n, 1), jnp.float32),
            pltpu.VMEM((l, n, 2 * _INNER + 3), jnp.bfloat16),
            pltpu.VMEM((l, n, _HEADS), jnp.float32),
            pltpu.VMEM((n, _HEADS), jnp.float32),
            pltpu.VMEM((n, _INNER), jnp.float32),
            pltpu.VMEM((n, 3), jnp.float32),
            pltpu.VMEM((n, _HEADS), jnp.float32),
            pltpu.VMEM((n, _HEADS), jnp.float32),
            pltpu.VMEM((n, _INNER), jnp.float32),
            pltpu.VMEM((3, n, _HEADS), jnp.float32),
        ],
    )(xyzs_f, xyzs_t, kv.reshape(b, l, n, 2 * _INNER),
      q.reshape(b, l, n, _INNER), wsp_tiled, W_out.T, b_out.reshape(1, dim))

    out = pl.pallas_call(
        _resid_body,
        grid=(b * l,),
        in_specs=[
            pl.BlockSpec((1, n, dim), lambda i: (i // l, 0, 0)),
            pl.BlockSpec((1, n, dim), lambda i: (i, 0, 0)),
        ],
        out_specs=pl.BlockSpec((1, n, dim), lambda i: (i, 0, 0)),
        out_shape=jax.ShapeDtypeStruct((b * l, n, dim), jnp.float32),
    )(g_out, ff)
    return out.reshape(b, l, n, dim)
